# Initial kernel scaffold; baseline (speedup 1.0000x reference)
import jax
import jax.numpy as jnp
from jax.experimental import pallas as pl


def kernel(x, edge_index, batch, W1, b1, W2, b2, W3, b3, W4, b4, W5, b5, W6, b6, gn1_w, gn1_b, gn1_ms, gn2_w, gn2_b, gn2_ms, gn3_w, gn3_b, gn3_ms, gn6_w, gn6_b, gn6_ms, lin1_W, lin1_b, lin2_W, lin2_b):
    raise NotImplementedError("write your pallas kernel here")


# speedup vs baseline: 2.7090x; 2.7090x over previous
import functools

import jax
import jax.numpy as jnp
from jax import lax
from jax.experimental import pallas as pl
from jax.experimental.pallas import tpu as pltpu
from jax.experimental.pallas import tpu_sc as plsc

N_NODES = 10000
N_PAD = 10112
E_EDGES = 320000
CHUNK = 128
E_PAD = 323584
ROWS_PER_TILE = N_PAD // 16


@functools.lru_cache(maxsize=None)
def _make_spmm(dh):
    mesh = plsc.VectorSubcoreMesh(core_axis_name="c", subcore_axis_name="s")
    ept = E_PAD // 16
    n_chunks = ept // CHUNK

    @functools.partial(
        pl.kernel,
        mesh=mesh,
        out_type=jax.ShapeDtypeStruct((2 * N_PAD, dh), jnp.float32),
        compiler_params=pltpu.CompilerParams(use_tc_tiling_on_sc=False),
        scratch_types=[
            pltpu.VMEM((CHUNK,), jnp.int32),
            pltpu.VMEM((CHUNK,), jnp.int32),
            pltpu.VMEM((CHUNK, dh), jnp.float32),
            pltpu.VMEM((ROWS_PER_TILE, dh), jnp.float32),
            pltpu.VMEM_SHARED((N_PAD, dh), jnp.float32),
            pltpu.SemaphoreType.DMA,
        ],
    )
    def spmm(g_hbm, row2_hbm, col_hbm, zero_hbm, out_hbm,
             gidx, sidx, rows, zbuf, acc, sem):
        c = lax.axis_index("c")
        s = lax.axis_index("s")

        pltpu.sync_copy(zero_hbm, zbuf)
        r0 = s * ROWS_PER_TILE
        pltpu.sync_copy(zbuf, acc.at[pl.ds(r0, ROWS_PER_TILE)])
        plsc.subcore_barrier()

        ebase = s * ept

        def body(i, _):
            base = ebase + i * CHUNK
            pltpu.sync_copy(row2_hbm.at[pl.ds(c * E_PAD + base, CHUNK)], gidx)
            pltpu.sync_copy(col_hbm.at[pl.ds(base, CHUNK)], sidx)
            pltpu.async_copy(g_hbm.at[gidx], rows, sem).wait()
            pltpu.sync_copy(rows, acc.at[sidx], add=True)
            return 0

        lax.fori_loop(0, n_chunks, body, 0)
        plsc.subcore_barrier()

        pltpu.sync_copy(acc.at[pl.ds(r0, ROWS_PER_TILE)], zbuf)
        pltpu.sync_copy(zbuf, out_hbm.at[pl.ds(c * N_PAD + r0, ROWS_PER_TILE)])

    return spmm


def _spmm(g2, row2, col, dh):
    zero = jnp.zeros((ROWS_PER_TILE, dh), jnp.float32)
    out = _make_spmm(dh)(g2, row2, col, zero)
    return out


@functools.lru_cache(maxsize=None)
def _make_deg():
    mesh = plsc.VectorSubcoreMesh(core_axis_name="c", subcore_axis_name="s")
    ept = E_PAD // 32
    n_chunks = ept // CHUNK

    @functools.partial(
        pl.kernel,
        mesh=mesh,
        out_type=jax.ShapeDtypeStruct((2 * N_PAD, 16), jnp.float32),
        compiler_params=pltpu.CompilerParams(use_tc_tiling_on_sc=False),
        scratch_types=[
            pltpu.VMEM((CHUNK,), jnp.int32),
            pltpu.VMEM((CHUNK, 16), jnp.float32),
            pltpu.VMEM((ROWS_PER_TILE, 16), jnp.float32),
            pltpu.VMEM_SHARED((N_PAD, 16), jnp.float32),
        ],
    )
    def deg(row_hbm, ones_hbm, zero_hbm, out_hbm, sidx, ones, zbuf, acc):
        c = lax.axis_index("c")
        s = lax.axis_index("s")

        pltpu.sync_copy(ones_hbm, ones)
        pltpu.sync_copy(zero_hbm, zbuf)
        r0 = s * ROWS_PER_TILE
        pltpu.sync_copy(zbuf, acc.at[pl.ds(r0, ROWS_PER_TILE)])
        plsc.subcore_barrier()

        ebase = (c * 16 + s) * ept

        def body(i, _):
            base = ebase + i * CHUNK
            pltpu.sync_copy(row_hbm.at[pl.ds(base, CHUNK)], sidx)
            pltpu.sync_copy(ones, acc.at[sidx], add=True)
            return 0

        lax.fori_loop(0, n_chunks, body, 0)
        plsc.subcore_barrier()

        pltpu.sync_copy(acc.at[pl.ds(r0, ROWS_PER_TILE)], zbuf)
        pltpu.sync_copy(zbuf, out_hbm.at[pl.ds(c * N_PAD + r0, ROWS_PER_TILE)])

    return deg


def _degree(row_pad):
    ones = jnp.ones((CHUNK, 16), jnp.float32)
    zero = jnp.zeros((ROWS_PER_TILE, 16), jnp.float32)
    out = _make_deg()(row_pad, ones, zero)
    out = out.reshape(2, N_PAD, 16)
    return out[0, :, 0] + out[1, :, 0]


_NB = 4
_RB = N_PAD // _NB


def _mask_rows(y, i):
    rid = i * _RB + lax.broadcasted_iota(jnp.int32, y.shape, 0)
    return jnp.where(rid < N_NODES, y, 0.0)


def _mm(x, w, b=None, acc=None, act=None):
    n, fi = x.shape
    fo = w.shape[1]

    def body(*refs):
        i = pl.program_id(0)
        xr = refs[0]
        wr = refs[1]
        k = 2
        y = jnp.dot(xr[...], wr[...], preferred_element_type=jnp.float32)
        if b is not None:
            y = y + refs[k][...]
            k += 1
        if acc is not None:
            y = y + refs[k][...]
            k += 1
        if act == "relu":
            y = jnp.maximum(y, 0.0)
        refs[-1][...] = _mask_rows(y, i)

    in_specs = [
        pl.BlockSpec((_RB, fi), lambda i: (i, 0)),
        pl.BlockSpec((fi, fo), lambda i: (0, 0)),
    ]
    args = [x, w]
    if b is not None:
        in_specs.append(pl.BlockSpec((1, fo), lambda i: (0, 0)))
        args.append(b.reshape(1, fo))
    if acc is not None:
        in_specs.append(pl.BlockSpec((_RB, fo), lambda i: (i, 0)))
        args.append(acc)
    return pl.pallas_call(
        body,
        grid=(_NB,),
        in_specs=in_specs,
        out_specs=pl.BlockSpec((_RB, fo), lambda i: (i, 0)),
        out_shape=jax.ShapeDtypeStruct((n, fo), jnp.float32),
    )(*args)


def _prescale(x, dis):
    n, f = x.shape
    fq = f // 4

    def body(xr, dr, outr):
        y = dr[...] * xr[...]
        outr[...] = jnp.stack([y[:, q * fq:(q + 1) * fq] for q in range(4)])

    out = pl.pallas_call(
        body,
        grid=(_NB,),
        in_specs=[
            pl.BlockSpec((_RB, f), lambda i: (i, 0)),
            pl.BlockSpec((_RB, 1), lambda i: (i, 0)),
        ],
        out_specs=pl.BlockSpec((4, _RB, fq), lambda i: (0, i, 0)),
        out_shape=jax.ShapeDtypeStruct((4, n, fq), jnp.float32),
    )(x, dis)
    return out[0:2].reshape(2 * n, fq), out[2:4].reshape(2 * n, fq)


def _combine(aa, ab, dis, alpha, prev=None):
    fq = aa.shape[1]
    a3 = aa.reshape(2, N_PAD, fq)
    b3 = ab.reshape(2, N_PAD, fq)

    def body(*refs):
        ar, br, dr = refs[0], refs[1], refs[2]
        y = alpha * dr[...] * jnp.concatenate(
            [ar[0], ar[1], br[0], br[1]], axis=1)
        if prev is not None:
            y = y - refs[3][...]
        refs[-1][...] = y

    in_specs = [
        pl.BlockSpec((2, _RB, fq), lambda i: (0, i, 0)),
        pl.BlockSpec((2, _RB, fq), lambda i: (0, i, 0)),
        pl.BlockSpec((_RB, 1), lambda i: (i, 0)),
    ]
    args = [a3, b3, dis]
    if prev is not None:
        in_specs.append(pl.BlockSpec((_RB, 4 * fq), lambda i: (i, 0)))
        args.append(prev)
    return pl.pallas_call(
        body,
        grid=(_NB,),
        in_specs=in_specs,
        out_specs=pl.BlockSpec((_RB, 4 * fq), lambda i: (i, 0)),
        out_shape=jax.ShapeDtypeStruct((N_PAD, 4 * fq), jnp.float32),
    )(*args)


def _colstats(x):
    n, f = x.shape

    def body(xr, outr):
        i = pl.program_id(0)
        xv = xr[...]
        s1 = jnp.sum(xv, axis=0)
        s2 = jnp.sum(xv * xv, axis=0)
        part = jnp.concatenate(
            [s1[None], s2[None], jnp.zeros((6, f), jnp.float32)], axis=0)

        @pl.when(i == 0)
        def _():
            outr[...] = part

        @pl.when(i > 0)
        def _():
            outr[...] = outr[...] + part

    return pl.pallas_call(
        body,
        grid=(_NB,),
        in_specs=[pl.BlockSpec((_RB, f), lambda i: (i, 0))],
        out_specs=pl.BlockSpec((8, f), lambda i: (0, 0)),
        out_shape=jax.ShapeDtypeStruct((8, f), jnp.float32),
    )(x)


def _gnorm_apply(x, scale, shift, leaky):
    n, f = x.shape

    def body(xr, ar, cr, outr):
        i = pl.program_id(0)
        y = xr[...] * ar[...] + cr[...]
        if leaky:
            y = jnp.where(y > 0, y, 0.2 * y)
        outr[...] = _mask_rows(y, i)

    return pl.pallas_call(
        body,
        grid=(_NB,),
        in_specs=[
            pl.BlockSpec((_RB, f), lambda i: (i, 0)),
            pl.BlockSpec((1, f), lambda i: (0, 0)),
            pl.BlockSpec((1, f), lambda i: (0, 0)),
        ],
        out_specs=pl.BlockSpec((_RB, f), lambda i: (i, 0)),
        out_shape=jax.ShapeDtypeStruct((n, f), jnp.float32),
    )(x, scale.reshape(1, f), shift.reshape(1, f))


def _head(pooled, w1, b1, w2, b2):
    def body(pr, w1r, b1r, w2r, b2r, outr):
        t = jnp.tanh(jnp.dot(pr[...], w1r[...],
                             preferred_element_type=jnp.float32) + b1r[...])
        outr[...] = jnp.dot(t, w2r[...],
                            preferred_element_type=jnp.float32) + b2r[...]

    return pl.pallas_call(
        body,
        out_shape=jax.ShapeDtypeStruct((8, w2.shape[1]), jnp.float32),
    )(pooled, w1, b1.reshape(1, -1), w2, b2.reshape(1, -1))


def _pad_to(v, m):
    r = (-v.shape[-1]) % m
    if r == 0:
        return v
    width = [(0, 0)] * (v.ndim - 1) + [(0, r)]
    return jnp.pad(v, width)


def _propagate(x, dis, row2, col):
    ga, gb = _prescale(x, dis)
    fq = ga.shape[1]
    return _spmm(ga, row2, col, fq), _spmm(gb, row2, col, fq)


def _cheb_layer(h, dis, row2, col, w, b, act=None):
    k = w.shape[0]
    out = _mm(h, w[0], b=b, act=act if k == 1 else None)
    if k == 1:
        return out
    aa, ab = _propagate(h, dis, row2, col)
    tx1 = _combine(aa, ab, dis, -1.0)
    out = _mm(tx1, w[1], acc=out, act=act if k == 2 else None)
    tx0 = h
    for j in range(2, k):
        aa, ab = _propagate(tx1, dis, row2, col)
        tx2 = _combine(aa, ab, dis, -2.0, prev=tx0)
        out = _mm(tx2, w[j], acc=out, act=act if j == k - 1 else None)
        tx0, tx1 = tx1, tx2
    return out


def _gnorm(h, gw, gb, gms, leaky=True):
    f = h.shape[1]
    st = _colstats(h)
    s1, s2 = st[0], st[1]
    mean = s1 / N_NODES
    ex2 = s2 / N_NODES
    var = ex2 - 2.0 * gms * mean * mean + gms * gms * mean * mean
    inv = 1.0 / jnp.sqrt(var + 1e-5)
    scale = inv * gw
    shift = gb - gms * mean * scale
    return _gnorm_apply(h, scale, shift, leaky)


def kernel(x, edge_index, batch, W1, b1, W2, b2, W3, b3, W4, b4, W5, b5,
           W6, b6, gn1_w, gn1_b, gn1_ms, gn2_w, gn2_b, gn2_ms,
           gn3_w, gn3_b, gn3_ms, gn6_w, gn6_b, gn6_ms,
           lin1_W, lin1_b, lin2_W, lin2_b):
    f32 = jnp.float32

    row = edge_index[0]
    col = edge_index[1]
    pad = jnp.full((E_PAD - E_EDGES,), N_NODES, jnp.int32)
    rowp = jnp.concatenate([row, pad])
    colp = jnp.concatenate([col, pad])
    row2 = jnp.concatenate([rowp, rowp + N_PAD])

    deg = _degree(rowp)[:N_NODES]
    dis_n = jnp.where(deg > 0, 1.0 / jnp.sqrt(jnp.maximum(deg, 1.0)), 0.0)
    dis = jnp.zeros((N_PAD, 1), f32).at[:N_NODES, 0].set(dis_n)

    FW = 256
    h = jnp.pad(x, ((0, N_PAD - N_NODES), (0, FW - x.shape[1])))

    def padw(w, bb):
        w = _pad_to(w, FW)
        if w.shape[1] != FW:
            w = jnp.pad(w, ((0, 0), (0, FW - w.shape[1]), (0, 0)))
        return w, _pad_to(bb, FW)

    W1p, b1p = padw(W1, b1)
    W2p, b2p = padw(W2, b2)
    W3p, b3p = padw(W3, b3)
    W4p, b4p = padw(W4, b4)
    W5p, b5p = padw(W5, b5)
    W6p, b6p = padw(W6, b6)

    h = _cheb_layer(h, dis, row2, colp, W1p, b1p)
    h = _gnorm(h, _pad_to(gn1_w, FW), _pad_to(gn1_b, FW), _pad_to(gn1_ms, FW))
    h = _cheb_layer(h, dis, row2, colp, W2p, b2p)
    h = _gnorm(h, _pad_to(gn2_w, FW), _pad_to(gn2_b, FW), _pad_to(gn2_ms, FW))
    h = _cheb_layer(h, dis, row2, colp, W3p, b3p)
    h = _gnorm(h, _pad_to(gn3_w, FW), _pad_to(gn3_b, FW), _pad_to(gn3_ms, FW))
    h = _cheb_layer(h, dis, row2, colp, W4p, b4p, act="relu")
    h = _cheb_layer(h, dis, row2, colp, W5p, b5p, act="relu")
    h = _cheb_layer(h, dis, row2, colp, W6p, b6p)
    h = _gnorm(h, _pad_to(gn6_w, FW), _pad_to(gn6_b, FW), _pad_to(gn6_ms, FW))

    st = _colstats(h)
    pooled = st[0] / N_NODES
    pooled8 = jnp.broadcast_to(pooled[None], (8, h.shape[1]))
    lin1p = jnp.pad(lin1_W, ((0, FW - lin1_W.shape[0]), (0, 0)))
    out = _head(pooled8, lin1p, lin1_b, lin2_W, lin2_b)
    return out[0:1]


# speedup vs baseline: 3.1770x; 1.1727x over previous
import functools

import jax
import jax.numpy as jnp
from jax import lax
from jax.experimental import pallas as pl
from jax.experimental.pallas import tpu as pltpu
from jax.experimental.pallas import tpu_sc as plsc

N_NODES = 10000
N_PAD = 10112
E_EDGES = 320000
CHUNK = 128
E_PAD = 327680
IBLK = 16
ROWS_PER_TILE = N_PAD // 16


@functools.lru_cache(maxsize=None)
def _make_spmm(dh):
    mesh = plsc.VectorSubcoreMesh(core_axis_name="c", subcore_axis_name="s")
    ept = E_PAD // 16
    n_chunks = ept // CHUNK

    @functools.partial(
        pl.kernel,
        mesh=mesh,
        out_type=jax.ShapeDtypeStruct((2 * N_PAD, dh), jnp.float32),
        compiler_params=pltpu.CompilerParams(use_tc_tiling_on_sc=False),
        scratch_types=[
            pltpu.VMEM((IBLK, CHUNK), jnp.int32),
            pltpu.VMEM((IBLK, CHUNK), jnp.int32),
            pltpu.VMEM((CHUNK, dh), jnp.float32),
            pltpu.VMEM((CHUNK, dh), jnp.float32),
            pltpu.VMEM((ROWS_PER_TILE, dh), jnp.float32),
            pltpu.VMEM_SHARED((N_PAD, dh), jnp.float32),
            pltpu.SemaphoreType.DMA,
            pltpu.SemaphoreType.DMA,
        ],
    )
    def spmm(g_hbm, row2_hbm, col_hbm, zero_hbm, out_hbm,
             gidx, sidx, rows0, rows1, zbuf, acc, sem0, sem1):
        c = lax.axis_index("c")
        s = lax.axis_index("s")

        crow = (c * E_PAD + s * ept) // CHUNK
        ccol = (s * ept) // CHUNK

        pltpu.sync_copy(zero_hbm, zbuf)
        r0 = s * ROWS_PER_TILE
        pltpu.sync_copy(zbuf, acc.at[pl.ds(r0, ROWS_PER_TILE)])
        plsc.subcore_barrier()

        def gather(k, rows, sem):
            return pltpu.make_async_copy(g_hbm.at[gidx.at[k]], rows, sem)

        def block(bi, _):
            pltpu.sync_copy(row2_hbm.at[pl.ds(crow + bi * IBLK, IBLK)], gidx)
            pltpu.sync_copy(col_hbm.at[pl.ds(ccol + bi * IBLK, IBLK)], sidx)
            gather(0, rows0, sem0).start()

            def body(j, _):
                k = 2 * j
                gather(k, rows0, sem0).wait()
                gather(k + 1, rows1, sem1).start()
                pltpu.sync_copy(rows0, acc.at[sidx.at[k]], add=True)
                gather(k + 1, rows1, sem1).wait()

                @pl.when(j + 1 < IBLK // 2)
                def _():
                    gather(k + 2, rows0, sem0).start()

                pltpu.sync_copy(rows1, acc.at[sidx.at[k + 1]], add=True)
                return 0

            lax.fori_loop(0, IBLK // 2, body, 0)
            return 0

        lax.fori_loop(0, n_chunks // IBLK, block, 0)
        plsc.subcore_barrier()

        pltpu.sync_copy(acc.at[pl.ds(r0, ROWS_PER_TILE)], zbuf)
        pltpu.sync_copy(zbuf, out_hbm.at[pl.ds(c * N_PAD + r0, ROWS_PER_TILE)])

    return spmm


def _spmm(g2, row2_2d, col_2d, dh):
    zero = jnp.zeros((ROWS_PER_TILE, dh), jnp.float32)
    out = _make_spmm(dh)(g2, row2_2d, col_2d, zero)
    return out


@functools.lru_cache(maxsize=None)
def _make_deg():
    mesh = plsc.VectorSubcoreMesh(core_axis_name="c", subcore_axis_name="s")
    ept = E_PAD // 32
    n_chunks = ept // CHUNK

    @functools.partial(
        pl.kernel,
        mesh=mesh,
        out_type=jax.ShapeDtypeStruct((2 * N_PAD, 16), jnp.float32),
        compiler_params=pltpu.CompilerParams(use_tc_tiling_on_sc=False),
        scratch_types=[
            pltpu.VMEM((IBLK, CHUNK), jnp.int32),
            pltpu.VMEM((CHUNK, 16), jnp.float32),
            pltpu.VMEM((ROWS_PER_TILE, 16), jnp.float32),
            pltpu.VMEM_SHARED((N_PAD, 16), jnp.float32),
        ],
    )
    def deg(row_hbm, ones_hbm, zero_hbm, out_hbm, sidx, ones, zbuf, acc):
        c = lax.axis_index("c")
        s = lax.axis_index("s")

        cbase = ((c * 16 + s) * ept) // CHUNK
        pltpu.sync_copy(ones_hbm, ones)
        pltpu.sync_copy(zero_hbm, zbuf)
        r0 = s * ROWS_PER_TILE
        pltpu.sync_copy(zbuf, acc.at[pl.ds(r0, ROWS_PER_TILE)])
        plsc.subcore_barrier()

        def block(bi, _):
            pltpu.sync_copy(row_hbm.at[pl.ds(cbase + bi * IBLK, IBLK)], sidx)

            def body(i, _):
                pltpu.sync_copy(ones, acc.at[sidx.at[i]], add=True)
                return 0

            lax.fori_loop(0, IBLK, body, 0)
            return 0

        lax.fori_loop(0, n_chunks // IBLK, block, 0)
        plsc.subcore_barrier()

        pltpu.sync_copy(acc.at[pl.ds(r0, ROWS_PER_TILE)], zbuf)
        pltpu.sync_copy(zbuf, out_hbm.at[pl.ds(c * N_PAD + r0, ROWS_PER_TILE)])

    return deg


def _degree(row_2d):
    ones = jnp.ones((CHUNK, 16), jnp.float32)
    zero = jnp.zeros((ROWS_PER_TILE, 16), jnp.float32)
    out = _make_deg()(row_2d, ones, zero)
    out = out.reshape(2, N_PAD, 16)
    return out[0, :, 0] + out[1, :, 0]


_NB = 4
_RB = N_PAD // _NB


def _mask_rows(y, i):
    rid = i * _RB + lax.broadcasted_iota(jnp.int32, y.shape, 0)
    return jnp.where(rid < N_NODES, y, 0.0)


def _mm(x, w, b=None, acc=None, act=None):
    n, fi = x.shape
    fo = w.shape[1]

    def body(*refs):
        i = pl.program_id(0)
        xr = refs[0]
        wr = refs[1]
        k = 2
        y = jnp.dot(xr[...], wr[...], preferred_element_type=jnp.float32)
        if b is not None:
            y = y + refs[k][...]
            k += 1
        if acc is not None:
            y = y + refs[k][...]
            k += 1
        if act == "relu":
            y = jnp.maximum(y, 0.0)
        refs[-1][...] = _mask_rows(y, i)

    in_specs = [
        pl.BlockSpec((_RB, fi), lambda i: (i, 0)),
        pl.BlockSpec((fi, fo), lambda i: (0, 0)),
    ]
    args = [x, w]
    if b is not None:
        in_specs.append(pl.BlockSpec((1, fo), lambda i: (0, 0)))
        args.append(b.reshape(1, fo))
    if acc is not None:
        in_specs.append(pl.BlockSpec((_RB, fo), lambda i: (i, 0)))
        args.append(acc)
    return pl.pallas_call(
        body,
        grid=(_NB,),
        in_specs=in_specs,
        out_specs=pl.BlockSpec((_RB, fo), lambda i: (i, 0)),
        out_shape=jax.ShapeDtypeStruct((n, fo), jnp.float32),
    )(*args)


def _prescale(x, dis):
    n, f = x.shape
    fq = f // 4

    def body(xr, dr, outr):
        y = dr[...] * xr[...]
        outr[...] = jnp.stack([y[:, q * fq:(q + 1) * fq] for q in range(4)])

    out = pl.pallas_call(
        body,
        grid=(_NB,),
        in_specs=[
            pl.BlockSpec((_RB, f), lambda i: (i, 0)),
            pl.BlockSpec((_RB, 1), lambda i: (i, 0)),
        ],
        out_specs=pl.BlockSpec((4, _RB, fq), lambda i: (0, i, 0)),
        out_shape=jax.ShapeDtypeStruct((4, n, fq), jnp.float32),
    )(x, dis)
    return out[0:2].reshape(2 * n, fq), out[2:4].reshape(2 * n, fq)


def _combine(aa, ab, dis, alpha, prev=None):
    fq = aa.shape[1]
    a3 = aa.reshape(2, N_PAD, fq)
    b3 = ab.reshape(2, N_PAD, fq)

    def body(*refs):
        ar, br, dr = refs[0], refs[1], refs[2]
        y = alpha * dr[...] * jnp.concatenate(
            [ar[0], ar[1], br[0], br[1]], axis=1)
        if prev is not None:
            y = y - refs[3][...]
        refs[-1][...] = y

    in_specs = [
        pl.BlockSpec((2, _RB, fq), lambda i: (0, i, 0)),
        pl.BlockSpec((2, _RB, fq), lambda i: (0, i, 0)),
        pl.BlockSpec((_RB, 1), lambda i: (i, 0)),
    ]
    args = [a3, b3, dis]
    if prev is not None:
        in_specs.append(pl.BlockSpec((_RB, 4 * fq), lambda i: (i, 0)))
        args.append(prev)
    return pl.pallas_call(
        body,
        grid=(_NB,),
        in_specs=in_specs,
        out_specs=pl.BlockSpec((_RB, 4 * fq), lambda i: (i, 0)),
        out_shape=jax.ShapeDtypeStruct((N_PAD, 4 * fq), jnp.float32),
    )(*args)


def _colstats(x):
    n, f = x.shape

    def body(xr, outr):
        i = pl.program_id(0)
        xv = xr[...]
        s1 = jnp.sum(xv, axis=0)
        s2 = jnp.sum(xv * xv, axis=0)
        part = jnp.concatenate(
            [s1[None], s2[None], jnp.zeros((6, f), jnp.float32)], axis=0)

        @pl.when(i == 0)
        def _():
            outr[...] = part

        @pl.when(i > 0)
        def _():
            outr[...] = outr[...] + part

    return pl.pallas_call(
        body,
        grid=(_NB,),
        in_specs=[pl.BlockSpec((_RB, f), lambda i: (i, 0))],
        out_specs=pl.BlockSpec((8, f), lambda i: (0, 0)),
        out_shape=jax.ShapeDtypeStruct((8, f), jnp.float32),
    )(x)


def _gnorm_apply(x, scale, shift, leaky):
    n, f = x.shape

    def body(xr, ar, cr, outr):
        i = pl.program_id(0)
        y = xr[...] * ar[...] + cr[...]
        if leaky:
            y = jnp.where(y > 0, y, 0.2 * y)
        outr[...] = _mask_rows(y, i)

    return pl.pallas_call(
        body,
        grid=(_NB,),
        in_specs=[
            pl.BlockSpec((_RB, f), lambda i: (i, 0)),
            pl.BlockSpec((1, f), lambda i: (0, 0)),
            pl.BlockSpec((1, f), lambda i: (0, 0)),
        ],
        out_specs=pl.BlockSpec((_RB, f), lambda i: (i, 0)),
        out_shape=jax.ShapeDtypeStruct((n, f), jnp.float32),
    )(x, scale.reshape(1, f), shift.reshape(1, f))


def _head(pooled, w1, b1, w2, b2):
    def body(pr, w1r, b1r, w2r, b2r, outr):
        t = jnp.tanh(jnp.dot(pr[...], w1r[...],
                             preferred_element_type=jnp.float32) + b1r[...])
        outr[...] = jnp.dot(t, w2r[...],
                            preferred_element_type=jnp.float32) + b2r[...]

    return pl.pallas_call(
        body,
        out_shape=jax.ShapeDtypeStruct((8, w2.shape[1]), jnp.float32),
    )(pooled, w1, b1.reshape(1, -1), w2, b2.reshape(1, -1))


def _pad_to(v, m):
    r = (-v.shape[-1]) % m
    if r == 0:
        return v
    width = [(0, 0)] * (v.ndim - 1) + [(0, r)]
    return jnp.pad(v, width)


def _propagate(x, dis, row2, col):
    ga, gb = _prescale(x, dis)
    fq = ga.shape[1]
    return _spmm(ga, row2, col, fq), _spmm(gb, row2, col, fq)


def _cheb_layer(h, dis, row2, col, w, b, act=None):
    k = w.shape[0]
    out = _mm(h, w[0], b=b, act=act if k == 1 else None)
    if k == 1:
        return out
    aa, ab = _propagate(h, dis, row2, col)
    tx1 = _combine(aa, ab, dis, -1.0)
    out = _mm(tx1, w[1], acc=out, act=act if k == 2 else None)
    tx0 = h
    for j in range(2, k):
        aa, ab = _propagate(tx1, dis, row2, col)
        tx2 = _combine(aa, ab, dis, -2.0, prev=tx0)
        out = _mm(tx2, w[j], acc=out, act=act if j == k - 1 else None)
        tx0, tx1 = tx1, tx2
    return out


def _gnorm(h, gw, gb, gms, leaky=True):
    f = h.shape[1]
    st = _colstats(h)
    s1, s2 = st[0], st[1]
    mean = s1 / N_NODES
    ex2 = s2 / N_NODES
    var = ex2 - 2.0 * gms * mean * mean + gms * gms * mean * mean
    inv = 1.0 / jnp.sqrt(var + 1e-5)
    scale = inv * gw
    shift = gb - gms * mean * scale
    return _gnorm_apply(h, scale, shift, leaky)


def kernel(x, edge_index, batch, W1, b1, W2, b2, W3, b3, W4, b4, W5, b5,
           W6, b6, gn1_w, gn1_b, gn1_ms, gn2_w, gn2_b, gn2_ms,
           gn3_w, gn3_b, gn3_ms, gn6_w, gn6_b, gn6_ms,
           lin1_W, lin1_b, lin2_W, lin2_b):
    f32 = jnp.float32

    row = edge_index[0]
    col = edge_index[1]
    pad = jnp.full((E_PAD - E_EDGES,), N_NODES, jnp.int32)
    rowp = jnp.concatenate([row, pad])
    colp = jnp.concatenate([col, pad]).reshape(-1, CHUNK)
    row2 = jnp.concatenate([rowp, rowp + N_PAD]).reshape(-1, CHUNK)

    deg = _degree(rowp.reshape(-1, CHUNK))[:N_NODES]
    dis_n = jnp.where(deg > 0, 1.0 / jnp.sqrt(jnp.maximum(deg, 1.0)), 0.0)
    dis = jnp.zeros((N_PAD, 1), f32).at[:N_NODES, 0].set(dis_n)

    FW = 256
    h = jnp.pad(x, ((0, N_PAD - N_NODES), (0, FW - x.shape[1])))

    def padw(w, bb):
        w = _pad_to(w, FW)
        if w.shape[1] != FW:
            w = jnp.pad(w, ((0, 0), (0, FW - w.shape[1]), (0, 0)))
        return w, _pad_to(bb, FW)

    W1p, b1p = padw(W1, b1)
    W2p, b2p = padw(W2, b2)
    W3p, b3p = padw(W3, b3)
    W4p, b4p = padw(W4, b4)
    W5p, b5p = padw(W5, b5)
    W6p, b6p = padw(W6, b6)

    h = _cheb_layer(h, dis, row2, colp, W1p, b1p)
    h = _gnorm(h, _pad_to(gn1_w, FW), _pad_to(gn1_b, FW), _pad_to(gn1_ms, FW))
    h = _cheb_layer(h, dis, row2, colp, W2p, b2p)
    h = _gnorm(h, _pad_to(gn2_w, FW), _pad_to(gn2_b, FW), _pad_to(gn2_ms, FW))
    h = _cheb_layer(h, dis, row2, colp, W3p, b3p)
    h = _gnorm(h, _pad_to(gn3_w, FW), _pad_to(gn3_b, FW), _pad_to(gn3_ms, FW))
    h = _cheb_layer(h, dis, row2, colp, W4p, b4p, act="relu")
    h = _cheb_layer(h, dis, row2, colp, W5p, b5p, act="relu")
    h = _cheb_layer(h, dis, row2, colp, W6p, b6p)
    h = _gnorm(h, _pad_to(gn6_w, FW), _pad_to(gn6_b, FW), _pad_to(gn6_ms, FW))

    st = _colstats(h)
    pooled = st[0] / N_NODES
    pooled8 = jnp.broadcast_to(pooled[None], (8, h.shape[1]))
    lin1p = jnp.pad(lin1_W, ((0, FW - lin1_W.shape[0]), (0, 0)))
    out = _head(pooled8, lin1p, lin1_b, lin2_W, lin2_b)
    return out[0:1]


# speedup vs baseline: 3.6727x; 1.1560x over previous
import functools

import jax
import jax.numpy as jnp
from jax import lax
from jax.experimental import pallas as pl
from jax.experimental.pallas import tpu as pltpu
from jax.experimental.pallas import tpu_sc as plsc

N_NODES = 10000
N_PAD = 10112
E_EDGES = 320000
CHUNK = 128
E_PAD = 327680
IBLK = 16
ROWS_PER_TILE = N_PAD // 16


@functools.lru_cache(maxsize=None)
def _make_spmm(dh):
    mesh = plsc.VectorSubcoreMesh(core_axis_name="c", subcore_axis_name="s")
    ept = E_PAD // 16
    n_chunks = ept // CHUNK
    NBUF = 4

    @functools.partial(
        pl.kernel,
        mesh=mesh,
        out_type=jax.ShapeDtypeStruct((4 * N_PAD, dh), jnp.float32),
        compiler_params=pltpu.CompilerParams(use_tc_tiling_on_sc=False),
        scratch_types=[
            pltpu.VMEM((IBLK, CHUNK), jnp.int32),
            pltpu.VMEM((IBLK, CHUNK), jnp.int32),
            pltpu.VMEM((NBUF, CHUNK, dh), jnp.float32),
            pltpu.VMEM((ROWS_PER_TILE, dh), jnp.float32),
            pltpu.VMEM_SHARED((N_PAD, dh), jnp.float32),
            pltpu.SemaphoreType.DMA,
            pltpu.SemaphoreType.DMA,
            pltpu.SemaphoreType.DMA,
            pltpu.SemaphoreType.DMA,
        ],
    )
    def spmm(g_hbm, row4_hbm, col_hbm, zero_hbm, out_hbm,
             gidx, sidx, rows, zbuf, acc, *sems):
        c = lax.axis_index("c")
        s = lax.axis_index("s")
        r0 = s * ROWS_PER_TILE
        ccol = (s * ept) // CHUNK

        def gather(k, b):
            return pltpu.make_async_copy(
                g_hbm.at[gidx.at[k % IBLK]], rows.at[b], sems[b])

        for q in range(2):
            pltpu.sync_copy(zero_hbm, zbuf)
            pltpu.sync_copy(zbuf, acc.at[pl.ds(r0, ROWS_PER_TILE)])
            plsc.subcore_barrier()

            crow = ((q * 2 + c) * E_PAD + s * ept) // CHUNK

            def block(bi, _):
                pltpu.sync_copy(row4_hbm.at[pl.ds(crow + bi * IBLK, IBLK)],
                                gidx)
                pltpu.sync_copy(col_hbm.at[pl.ds(ccol + bi * IBLK, IBLK)],
                                sidx)
                for b in range(NBUF - 1):
                    gather(b, b).start()

                def body(j, _):
                    k = NBUF * j
                    for b in range(NBUF):
                        gather(k + b, b).wait()

                        @pl.when(k + b + NBUF - 1 < IBLK)
                        def _():
                            gather(k + b + NBUF - 1, (b + NBUF - 1) % NBUF
                                   ).start()

                        pltpu.sync_copy(rows.at[b],
                                        acc.at[sidx.at[k + b]], add=True)
                    return 0

                lax.fori_loop(0, IBLK // NBUF, body, 0)
                return 0

            lax.fori_loop(0, n_chunks // IBLK, block, 0)
            plsc.subcore_barrier()

            pltpu.sync_copy(acc.at[pl.ds(r0, ROWS_PER_TILE)], zbuf)
            pltpu.sync_copy(
                zbuf, out_hbm.at[pl.ds((q * 2 + c) * N_PAD + r0,
                                       ROWS_PER_TILE)])
            plsc.subcore_barrier()

    return spmm


def _spmm(g4, row4_2d, col_2d, dh):
    zero = jnp.zeros((ROWS_PER_TILE, dh), jnp.float32)
    out = _make_spmm(dh)(g4, row4_2d, col_2d, zero)
    return out[:2 * N_PAD], out[2 * N_PAD:]


@functools.lru_cache(maxsize=None)
def _make_deg():
    mesh = plsc.VectorSubcoreMesh(core_axis_name="c", subcore_axis_name="s")
    ept = E_PAD // 32
    n_chunks = ept // CHUNK

    @functools.partial(
        pl.kernel,
        mesh=mesh,
        out_type=jax.ShapeDtypeStruct((2 * N_PAD, 16), jnp.float32),
        compiler_params=pltpu.CompilerParams(use_tc_tiling_on_sc=False),
        scratch_types=[
            pltpu.VMEM((IBLK, CHUNK), jnp.int32),
            pltpu.VMEM((CHUNK, 16), jnp.float32),
            pltpu.VMEM((ROWS_PER_TILE, 16), jnp.float32),
            pltpu.VMEM_SHARED((N_PAD, 16), jnp.float32),
        ],
    )
    def deg(row_hbm, ones_hbm, zero_hbm, out_hbm, sidx, ones, zbuf, acc):
        c = lax.axis_index("c")
        s = lax.axis_index("s")

        cbase = ((c * 16 + s) * ept) // CHUNK
        pltpu.sync_copy(ones_hbm, ones)
        pltpu.sync_copy(zero_hbm, zbuf)
        r0 = s * ROWS_PER_TILE
        pltpu.sync_copy(zbuf, acc.at[pl.ds(r0, ROWS_PER_TILE)])
        plsc.subcore_barrier()

        def block(bi, _):
            pltpu.sync_copy(row_hbm.at[pl.ds(cbase + bi * IBLK, IBLK)], sidx)

            def body(i, _):
                pltpu.sync_copy(ones, acc.at[sidx.at[i]], add=True)
                return 0

            lax.fori_loop(0, IBLK, body, 0)
            return 0

        lax.fori_loop(0, n_chunks // IBLK, block, 0)
        plsc.subcore_barrier()

        pltpu.sync_copy(acc.at[pl.ds(r0, ROWS_PER_TILE)], zbuf)
        pltpu.sync_copy(zbuf, out_hbm.at[pl.ds(c * N_PAD + r0, ROWS_PER_TILE)])

    return deg


def _degree(row_2d):
    ones = jnp.ones((CHUNK, 16), jnp.float32)
    zero = jnp.zeros((ROWS_PER_TILE, 16), jnp.float32)
    out = _make_deg()(row_2d, ones, zero)
    out = out.reshape(2, N_PAD, 16)
    return out[0, :, 0] + out[1, :, 0]


_NB = 4
_RB = N_PAD // _NB


def _mask_rows(y, i):
    rid = i * _RB + lax.broadcasted_iota(jnp.int32, y.shape, 0)
    return jnp.where(rid < N_NODES, y, 0.0)


def _mm(x, w, b=None, acc=None, act=None):
    n, fi = x.shape
    fo = w.shape[1]

    def body(*refs):
        i = pl.program_id(0)
        xr = refs[0]
        wr = refs[1]
        k = 2
        y = jnp.dot(xr[...], wr[...], preferred_element_type=jnp.float32)
        if b is not None:
            y = y + refs[k][...]
            k += 1
        if acc is not None:
            y = y + refs[k][...]
            k += 1
        if act == "relu":
            y = jnp.maximum(y, 0.0)
        refs[-1][...] = _mask_rows(y, i)

    in_specs = [
        pl.BlockSpec((_RB, fi), lambda i: (i, 0)),
        pl.BlockSpec((fi, fo), lambda i: (0, 0)),
    ]
    args = [x, w]
    if b is not None:
        in_specs.append(pl.BlockSpec((1, fo), lambda i: (0, 0)))
        args.append(b.reshape(1, fo))
    if acc is not None:
        in_specs.append(pl.BlockSpec((_RB, fo), lambda i: (i, 0)))
        args.append(acc)
    return pl.pallas_call(
        body,
        grid=(_NB,),
        in_specs=in_specs,
        out_specs=pl.BlockSpec((_RB, fo), lambda i: (i, 0)),
        out_shape=jax.ShapeDtypeStruct((n, fo), jnp.float32),
    )(*args)


def _prescale(x, dis):
    n, f = x.shape
    fq = f // 4

    def body(xr, dr, outr):
        y = dr[...] * xr[...]
        outr[...] = jnp.stack([y[:, q * fq:(q + 1) * fq] for q in range(4)])

    out = pl.pallas_call(
        body,
        grid=(_NB,),
        in_specs=[
            pl.BlockSpec((_RB, f), lambda i: (i, 0)),
            pl.BlockSpec((_RB, 1), lambda i: (i, 0)),
        ],
        out_specs=pl.BlockSpec((4, _RB, fq), lambda i: (0, i, 0)),
        out_shape=jax.ShapeDtypeStruct((4, n, fq), jnp.float32),
    )(x, dis)
    return out.reshape(4 * n, fq)


def _combine(aa, ab, dis, alpha, prev=None):
    fq = aa.shape[1]
    a3 = aa.reshape(2, N_PAD, fq)
    b3 = ab.reshape(2, N_PAD, fq)

    def body(*refs):
        ar, br, dr = refs[0], refs[1], refs[2]
        y = alpha * dr[...] * jnp.concatenate(
            [ar[0], ar[1], br[0], br[1]], axis=1)
        if prev is not None:
            y = y - refs[3][...]
        refs[-1][...] = y

    in_specs = [
        pl.BlockSpec((2, _RB, fq), lambda i: (0, i, 0)),
        pl.BlockSpec((2, _RB, fq), lambda i: (0, i, 0)),
        pl.BlockSpec((_RB, 1), lambda i: (i, 0)),
    ]
    args = [a3, b3, dis]
    if prev is not None:
        in_specs.append(pl.BlockSpec((_RB, 4 * fq), lambda i: (i, 0)))
        args.append(prev)
    return pl.pallas_call(
        body,
        grid=(_NB,),
        in_specs=in_specs,
        out_specs=pl.BlockSpec((_RB, 4 * fq), lambda i: (i, 0)),
        out_shape=jax.ShapeDtypeStruct((N_PAD, 4 * fq), jnp.float32),
    )(*args)


def _colstats(x):
    n, f = x.shape

    def body(xr, outr):
        i = pl.program_id(0)
        xv = xr[...]
        s1 = jnp.sum(xv, axis=0)
        s2 = jnp.sum(xv * xv, axis=0)
        part = jnp.concatenate(
            [s1[None], s2[None], jnp.zeros((6, f), jnp.float32)], axis=0)

        @pl.when(i == 0)
        def _():
            outr[...] = part

        @pl.when(i > 0)
        def _():
            outr[...] = outr[...] + part

    return pl.pallas_call(
        body,
        grid=(_NB,),
        in_specs=[pl.BlockSpec((_RB, f), lambda i: (i, 0))],
        out_specs=pl.BlockSpec((8, f), lambda i: (0, 0)),
        out_shape=jax.ShapeDtypeStruct((8, f), jnp.float32),
    )(x)


def _gnorm_apply(x, scale, shift, leaky):
    n, f = x.shape

    def body(xr, ar, cr, outr):
        i = pl.program_id(0)
        y = xr[...] * ar[...] + cr[...]
        if leaky:
            y = jnp.where(y > 0, y, 0.2 * y)
        outr[...] = _mask_rows(y, i)

    return pl.pallas_call(
        body,
        grid=(_NB,),
        in_specs=[
            pl.BlockSpec((_RB, f), lambda i: (i, 0)),
            pl.BlockSpec((1, f), lambda i: (0, 0)),
            pl.BlockSpec((1, f), lambda i: (0, 0)),
        ],
        out_specs=pl.BlockSpec((_RB, f), lambda i: (i, 0)),
        out_shape=jax.ShapeDtypeStruct((n, f), jnp.float32),
    )(x, scale.reshape(1, f), shift.reshape(1, f))


def _head(pooled, w1, b1, w2, b2):
    def body(pr, w1r, b1r, w2r, b2r, outr):
        t = jnp.tanh(jnp.dot(pr[...], w1r[...],
                             preferred_element_type=jnp.float32) + b1r[...])
        outr[...] = jnp.dot(t, w2r[...],
                            preferred_element_type=jnp.float32) + b2r[...]

    return pl.pallas_call(
        body,
        out_shape=jax.ShapeDtypeStruct((8, w2.shape[1]), jnp.float32),
    )(pooled, w1, b1.reshape(1, -1), w2, b2.reshape(1, -1))


def _pad_to(v, m):
    r = (-v.shape[-1]) % m
    if r == 0:
        return v
    width = [(0, 0)] * (v.ndim - 1) + [(0, r)]
    return jnp.pad(v, width)


def _propagate(x, dis, row4, col):
    g4 = _prescale(x, dis)
    return _spmm(g4, row4, col, g4.shape[1])


def _cheb_layer(h, dis, row2, col, w, b, act=None):
    k = w.shape[0]
    out = _mm(h, w[0], b=b, act=act if k == 1 else None)
    if k == 1:
        return out
    aa, ab = _propagate(h, dis, row2, col)
    tx1 = _combine(aa, ab, dis, -1.0)
    out = _mm(tx1, w[1], acc=out, act=act if k == 2 else None)
    tx0 = h
    for j in range(2, k):
        aa, ab = _propagate(tx1, dis, row2, col)
        tx2 = _combine(aa, ab, dis, -2.0, prev=tx0)
        out = _mm(tx2, w[j], acc=out, act=act if j == k - 1 else None)
        tx0, tx1 = tx1, tx2
    return out


def _gnorm(h, gw, gb, gms, leaky=True):
    f = h.shape[1]
    st = _colstats(h)
    s1, s2 = st[0], st[1]
    mean = s1 / N_NODES
    ex2 = s2 / N_NODES
    var = ex2 - 2.0 * gms * mean * mean + gms * gms * mean * mean
    inv = 1.0 / jnp.sqrt(var + 1e-5)
    scale = inv * gw
    shift = gb - gms * mean * scale
    return _gnorm_apply(h, scale, shift, leaky)


def kernel(x, edge_index, batch, W1, b1, W2, b2, W3, b3, W4, b4, W5, b5,
           W6, b6, gn1_w, gn1_b, gn1_ms, gn2_w, gn2_b, gn2_ms,
           gn3_w, gn3_b, gn3_ms, gn6_w, gn6_b, gn6_ms,
           lin1_W, lin1_b, lin2_W, lin2_b):
    f32 = jnp.float32

    row = edge_index[0]
    col = edge_index[1]
    pad = jnp.full((E_PAD - E_EDGES,), N_NODES, jnp.int32)
    rowp = jnp.concatenate([row, pad])
    colp = jnp.concatenate([col, pad]).reshape(-1, CHUNK)
    row2 = jnp.concatenate(
        [rowp + k * N_PAD for k in range(4)]).reshape(-1, CHUNK)

    deg = _degree(rowp.reshape(-1, CHUNK))[:N_NODES]
    dis_n = jnp.where(deg > 0, 1.0 / jnp.sqrt(jnp.maximum(deg, 1.0)), 0.0)
    dis = jnp.zeros((N_PAD, 1), f32).at[:N_NODES, 0].set(dis_n)

    FW = 256
    h = jnp.pad(x, ((0, N_PAD - N_NODES), (0, FW - x.shape[1])))

    def padw(w, bb):
        w = _pad_to(w, FW)
        if w.shape[1] != FW:
            w = jnp.pad(w, ((0, 0), (0, FW - w.shape[1]), (0, 0)))
        return w, _pad_to(bb, FW)

    W1p, b1p = padw(W1, b1)
    W2p, b2p = padw(W2, b2)
    W3p, b3p = padw(W3, b3)
    W4p, b4p = padw(W4, b4)
    W5p, b5p = padw(W5, b5)
    W6p, b6p = padw(W6, b6)

    h = _cheb_layer(h, dis, row2, colp, W1p, b1p)
    h = _gnorm(h, _pad_to(gn1_w, FW), _pad_to(gn1_b, FW), _pad_to(gn1_ms, FW))
    h = _cheb_layer(h, dis, row2, colp, W2p, b2p)
    h = _gnorm(h, _pad_to(gn2_w, FW), _pad_to(gn2_b, FW), _pad_to(gn2_ms, FW))
    h = _cheb_layer(h, dis, row2, colp, W3p, b3p)
    h = _gnorm(h, _pad_to(gn3_w, FW), _pad_to(gn3_b, FW), _pad_to(gn3_ms, FW))
    h = _cheb_layer(h, dis, row2, colp, W4p, b4p, act="relu")
    h = _cheb_layer(h, dis, row2, colp, W5p, b5p, act="relu")
    h = _cheb_layer(h, dis, row2, colp, W6p, b6p)
    h = _gnorm(h, _pad_to(gn6_w, FW), _pad_to(gn6_b, FW), _pad_to(gn6_ms, FW))

    st = _colstats(h)
    pooled = st[0] / N_NODES
    pooled8 = jnp.broadcast_to(pooled[None], (8, h.shape[1]))
    lin1p = jnp.pad(lin1_W, ((0, FW - lin1_W.shape[0]), (0, 0)))
    out = _head(pooled8, lin1p, lin1_b, lin2_W, lin2_b)
    return out[0:1]


# speedup vs baseline: 3.7751x; 1.0279x over previous
import functools

import jax
import jax.numpy as jnp
from jax import lax
from jax.experimental import pallas as pl
from jax.experimental.pallas import tpu as pltpu
from jax.experimental.pallas import tpu_sc as plsc

N_NODES = 10000
N_PAD = 10112
E_EDGES = 320000
CHUNK = 128
E_PAD = 327680
IBLK = 32
DEG_IBLK = 16
ROWS_PER_TILE = N_PAD // 16


@functools.lru_cache(maxsize=None)
def _make_spmm(dh):
    mesh = plsc.VectorSubcoreMesh(core_axis_name="c", subcore_axis_name="s")
    ept = E_PAD // 16
    n_chunks = ept // CHUNK
    NBUF = 4

    @functools.partial(
        pl.kernel,
        mesh=mesh,
        out_type=jax.ShapeDtypeStruct((4 * N_PAD, dh), jnp.float32),
        compiler_params=pltpu.CompilerParams(use_tc_tiling_on_sc=False),
        scratch_types=[
            pltpu.VMEM((IBLK, CHUNK), jnp.int32),
            pltpu.VMEM((IBLK, CHUNK), jnp.int32),
            pltpu.VMEM((NBUF, CHUNK, dh), jnp.float32),
            pltpu.VMEM((ROWS_PER_TILE, dh), jnp.float32),
            pltpu.VMEM_SHARED((N_PAD, dh), jnp.float32),
            pltpu.SemaphoreType.DMA,
            pltpu.SemaphoreType.DMA,
            pltpu.SemaphoreType.DMA,
            pltpu.SemaphoreType.DMA,
        ],
    )
    def spmm(g_hbm, row4_hbm, col_hbm, zero_hbm, out_hbm,
             gidx, sidx, rows, zbuf, acc, *sems):
        c = lax.axis_index("c")
        s = lax.axis_index("s")
        r0 = s * ROWS_PER_TILE
        ccol = (s * ept) // CHUNK

        def gather(k, b):
            return pltpu.make_async_copy(
                g_hbm.at[gidx.at[k % IBLK]], rows.at[b], sems[b])

        for q in range(2):
            pltpu.sync_copy(zero_hbm, zbuf)
            pltpu.sync_copy(zbuf, acc.at[pl.ds(r0, ROWS_PER_TILE)])
            plsc.subcore_barrier()

            crow = ((q * 2 + c) * E_PAD + s * ept) // CHUNK

            def block(bi, _):
                pltpu.sync_copy(row4_hbm.at[pl.ds(crow + bi * IBLK, IBLK)],
                                gidx)
                pltpu.sync_copy(col_hbm.at[pl.ds(ccol + bi * IBLK, IBLK)],
                                sidx)
                for b in range(NBUF - 1):
                    gather(b, b).start()

                def body(j, _):
                    k = NBUF * j
                    for b in range(NBUF):
                        gather(k + b, b).wait()

                        @pl.when(k + b + NBUF - 1 < IBLK)
                        def _():
                            gather(k + b + NBUF - 1, (b + NBUF - 1) % NBUF
                                   ).start()

                        pltpu.sync_copy(rows.at[b],
                                        acc.at[sidx.at[k + b]], add=True)
                    return 0

                lax.fori_loop(0, IBLK // NBUF, body, 0)
                return 0

            lax.fori_loop(0, n_chunks // IBLK, block, 0)
            plsc.subcore_barrier()

            pltpu.sync_copy(acc.at[pl.ds(r0, ROWS_PER_TILE)], zbuf)
            pltpu.sync_copy(
                zbuf, out_hbm.at[pl.ds((q * 2 + c) * N_PAD + r0,
                                       ROWS_PER_TILE)])
            plsc.subcore_barrier()

    return spmm


def _spmm(g4, row4_2d, col_2d, dh):
    zero = jnp.zeros((ROWS_PER_TILE, dh), jnp.float32)
    out = _make_spmm(dh)(g4, row4_2d, col_2d, zero)
    return out[:2 * N_PAD], out[2 * N_PAD:]


@functools.lru_cache(maxsize=None)
def _make_deg():
    mesh = plsc.VectorSubcoreMesh(core_axis_name="c", subcore_axis_name="s")
    ept = E_PAD // 32
    n_chunks = ept // CHUNK

    @functools.partial(
        pl.kernel,
        mesh=mesh,
        out_type=jax.ShapeDtypeStruct((2 * N_PAD, 16), jnp.float32),
        compiler_params=pltpu.CompilerParams(use_tc_tiling_on_sc=False),
        scratch_types=[
            pltpu.VMEM((DEG_IBLK, CHUNK), jnp.int32),
            pltpu.VMEM((CHUNK, 16), jnp.float32),
            pltpu.VMEM((ROWS_PER_TILE, 16), jnp.float32),
            pltpu.VMEM_SHARED((N_PAD, 16), jnp.float32),
        ],
    )
    def deg(row_hbm, ones_hbm, zero_hbm, out_hbm, sidx, ones, zbuf, acc):
        c = lax.axis_index("c")
        s = lax.axis_index("s")

        cbase = ((c * 16 + s) * ept) // CHUNK
        pltpu.sync_copy(ones_hbm, ones)
        pltpu.sync_copy(zero_hbm, zbuf)
        r0 = s * ROWS_PER_TILE
        pltpu.sync_copy(zbuf, acc.at[pl.ds(r0, ROWS_PER_TILE)])
        plsc.subcore_barrier()

        def block(bi, _):
            pltpu.sync_copy(row_hbm.at[pl.ds(cbase + bi * DEG_IBLK, DEG_IBLK)], sidx)

            def body(i, _):
                pltpu.sync_copy(ones, acc.at[sidx.at[i]], add=True)
                return 0

            lax.fori_loop(0, DEG_IBLK, body, 0)
            return 0

        lax.fori_loop(0, n_chunks // DEG_IBLK, block, 0)
        plsc.subcore_barrier()

        pltpu.sync_copy(acc.at[pl.ds(r0, ROWS_PER_TILE)], zbuf)
        pltpu.sync_copy(zbuf, out_hbm.at[pl.ds(c * N_PAD + r0, ROWS_PER_TILE)])

    return deg


def _degree(row_2d):
    ones = jnp.ones((CHUNK, 16), jnp.float32)
    zero = jnp.zeros((ROWS_PER_TILE, 16), jnp.float32)
    out = _make_deg()(row_2d, ones, zero)
    out = out.reshape(2, N_PAD, 16)
    return out[0, :, 0] + out[1, :, 0]


_NB = 4
_RB = N_PAD // _NB


def _mask_rows(y, i):
    rid = i * _RB + lax.broadcasted_iota(jnp.int32, y.shape, 0)
    return jnp.where(rid < N_NODES, y, 0.0)


def _mm(x, w, b=None, acc=None, act=None):
    n, fi = x.shape
    fo = w.shape[1]

    def body(*refs):
        i = pl.program_id(0)
        xr = refs[0]
        wr = refs[1]
        k = 2
        y = jnp.dot(xr[...], wr[...], preferred_element_type=jnp.float32)
        if b is not None:
            y = y + refs[k][...]
            k += 1
        if acc is not None:
            y = y + refs[k][...]
            k += 1
        if act == "relu":
            y = jnp.maximum(y, 0.0)
        refs[-1][...] = _mask_rows(y, i)

    in_specs = [
        pl.BlockSpec((_RB, fi), lambda i: (i, 0)),
        pl.BlockSpec((fi, fo), lambda i: (0, 0)),
    ]
    args = [x, w]
    if b is not None:
        in_specs.append(pl.BlockSpec((1, fo), lambda i: (0, 0)))
        args.append(b.reshape(1, fo))
    if acc is not None:
        in_specs.append(pl.BlockSpec((_RB, fo), lambda i: (i, 0)))
        args.append(acc)
    return pl.pallas_call(
        body,
        grid=(_NB,),
        in_specs=in_specs,
        out_specs=pl.BlockSpec((_RB, fo), lambda i: (i, 0)),
        out_shape=jax.ShapeDtypeStruct((n, fo), jnp.float32),
    )(*args)


def _prescale(x, dis):
    n, f = x.shape
    fq = f // 4

    def body(xr, dr, outr):
        y = dr[...] * xr[...]
        outr[...] = jnp.stack([y[:, q * fq:(q + 1) * fq] for q in range(4)])

    out = pl.pallas_call(
        body,
        grid=(_NB,),
        in_specs=[
            pl.BlockSpec((_RB, f), lambda i: (i, 0)),
            pl.BlockSpec((_RB, 1), lambda i: (i, 0)),
        ],
        out_specs=pl.BlockSpec((4, _RB, fq), lambda i: (0, i, 0)),
        out_shape=jax.ShapeDtypeStruct((4, n, fq), jnp.float32),
    )(x, dis)
    return out.reshape(4 * n, fq)


def _combine(aa, ab, dis, alpha, prev=None):
    fq = aa.shape[1]
    a3 = aa.reshape(2, N_PAD, fq)
    b3 = ab.reshape(2, N_PAD, fq)

    def body(*refs):
        ar, br, dr = refs[0], refs[1], refs[2]
        y = alpha * dr[...] * jnp.concatenate(
            [ar[0], ar[1], br[0], br[1]], axis=1)
        if prev is not None:
            y = y - refs[3][...]
        refs[-1][...] = y

    in_specs = [
        pl.BlockSpec((2, _RB, fq), lambda i: (0, i, 0)),
        pl.BlockSpec((2, _RB, fq), lambda i: (0, i, 0)),
        pl.BlockSpec((_RB, 1), lambda i: (i, 0)),
    ]
    args = [a3, b3, dis]
    if prev is not None:
        in_specs.append(pl.BlockSpec((_RB, 4 * fq), lambda i: (i, 0)))
        args.append(prev)
    return pl.pallas_call(
        body,
        grid=(_NB,),
        in_specs=in_specs,
        out_specs=pl.BlockSpec((_RB, 4 * fq), lambda i: (i, 0)),
        out_shape=jax.ShapeDtypeStruct((N_PAD, 4 * fq), jnp.float32),
    )(*args)


def _colstats(x):
    n, f = x.shape

    def body(xr, outr):
        i = pl.program_id(0)
        xv = xr[...]
        s1 = jnp.sum(xv, axis=0)
        s2 = jnp.sum(xv * xv, axis=0)
        part = jnp.concatenate(
            [s1[None], s2[None], jnp.zeros((6, f), jnp.float32)], axis=0)

        @pl.when(i == 0)
        def _():
            outr[...] = part

        @pl.when(i > 0)
        def _():
            outr[...] = outr[...] + part

    return pl.pallas_call(
        body,
        grid=(_NB,),
        in_specs=[pl.BlockSpec((_RB, f), lambda i: (i, 0))],
        out_specs=pl.BlockSpec((8, f), lambda i: (0, 0)),
        out_shape=jax.ShapeDtypeStruct((8, f), jnp.float32),
    )(x)


def _gnorm_apply(x, scale, shift, leaky):
    n, f = x.shape

    def body(xr, ar, cr, outr):
        i = pl.program_id(0)
        y = xr[...] * ar[...] + cr[...]
        if leaky:
            y = jnp.where(y > 0, y, 0.2 * y)
        outr[...] = _mask_rows(y, i)

    return pl.pallas_call(
        body,
        grid=(_NB,),
        in_specs=[
            pl.BlockSpec((_RB, f), lambda i: (i, 0)),
            pl.BlockSpec((1, f), lambda i: (0, 0)),
            pl.BlockSpec((1, f), lambda i: (0, 0)),
        ],
        out_specs=pl.BlockSpec((_RB, f), lambda i: (i, 0)),
        out_shape=jax.ShapeDtypeStruct((n, f), jnp.float32),
    )(x, scale.reshape(1, f), shift.reshape(1, f))


def _head(pooled, w1, b1, w2, b2):
    def body(pr, w1r, b1r, w2r, b2r, outr):
        t = jnp.tanh(jnp.dot(pr[...], w1r[...],
                             preferred_element_type=jnp.float32) + b1r[...])
        outr[...] = jnp.dot(t, w2r[...],
                            preferred_element_type=jnp.float32) + b2r[...]

    return pl.pallas_call(
        body,
        out_shape=jax.ShapeDtypeStruct((8, w2.shape[1]), jnp.float32),
    )(pooled, w1, b1.reshape(1, -1), w2, b2.reshape(1, -1))


def _pad_to(v, m):
    r = (-v.shape[-1]) % m
    if r == 0:
        return v
    width = [(0, 0)] * (v.ndim - 1) + [(0, r)]
    return jnp.pad(v, width)


def _propagate(x, dis, row4, col):
    g4 = _prescale(x, dis)
    return _spmm(g4, row4, col, g4.shape[1])


def _cheb_layer(h, dis, row2, col, w, b, act=None):
    k = w.shape[0]
    out = _mm(h, w[0], b=b, act=act if k == 1 else None)
    if k == 1:
        return out
    aa, ab = _propagate(h, dis, row2, col)
    tx1 = _combine(aa, ab, dis, -1.0)
    out = _mm(tx1, w[1], acc=out, act=act if k == 2 else None)
    tx0 = h
    for j in range(2, k):
        aa, ab = _propagate(tx1, dis, row2, col)
        tx2 = _combine(aa, ab, dis, -2.0, prev=tx0)
        out = _mm(tx2, w[j], acc=out, act=act if j == k - 1 else None)
        tx0, tx1 = tx1, tx2
    return out


def _gnorm(h, gw, gb, gms, leaky=True):
    f = h.shape[1]
    st = _colstats(h)
    s1, s2 = st[0], st[1]
    mean = s1 / N_NODES
    ex2 = s2 / N_NODES
    var = ex2 - 2.0 * gms * mean * mean + gms * gms * mean * mean
    inv = 1.0 / jnp.sqrt(var + 1e-5)
    scale = inv * gw
    shift = gb - gms * mean * scale
    return _gnorm_apply(h, scale, shift, leaky)


def kernel(x, edge_index, batch, W1, b1, W2, b2, W3, b3, W4, b4, W5, b5,
           W6, b6, gn1_w, gn1_b, gn1_ms, gn2_w, gn2_b, gn2_ms,
           gn3_w, gn3_b, gn3_ms, gn6_w, gn6_b, gn6_ms,
           lin1_W, lin1_b, lin2_W, lin2_b):
    f32 = jnp.float32

    row = edge_index[0]
    col = edge_index[1]
    pad = jnp.full((E_PAD - E_EDGES,), N_NODES, jnp.int32)
    rowp = jnp.concatenate([row, pad])
    colp = jnp.concatenate([col, pad]).reshape(-1, CHUNK)
    row2 = jnp.concatenate(
        [rowp + k * N_PAD for k in range(4)]).reshape(-1, CHUNK)

    deg = _degree(rowp.reshape(-1, CHUNK))[:N_NODES]
    dis_n = jnp.where(deg > 0, 1.0 / jnp.sqrt(jnp.maximum(deg, 1.0)), 0.0)
    dis = jnp.zeros((N_PAD, 1), f32).at[:N_NODES, 0].set(dis_n)

    FW = 256
    h = jnp.pad(x, ((0, N_PAD - N_NODES), (0, FW - x.shape[1])))

    def padw(w, bb):
        w = _pad_to(w, FW)
        if w.shape[1] != FW:
            w = jnp.pad(w, ((0, 0), (0, FW - w.shape[1]), (0, 0)))
        return w, _pad_to(bb, FW)

    W1p, b1p = padw(W1, b1)
    W2p, b2p = padw(W2, b2)
    W3p, b3p = padw(W3, b3)
    W4p, b4p = padw(W4, b4)
    W5p, b5p = padw(W5, b5)
    W6p, b6p = padw(W6, b6)

    h = _cheb_layer(h, dis, row2, colp, W1p, b1p)
    h = _gnorm(h, _pad_to(gn1_w, FW), _pad_to(gn1_b, FW), _pad_to(gn1_ms, FW))
    h = _cheb_layer(h, dis, row2, colp, W2p, b2p)
    h = _gnorm(h, _pad_to(gn2_w, FW), _pad_to(gn2_b, FW), _pad_to(gn2_ms, FW))
    h = _cheb_layer(h, dis, row2, colp, W3p, b3p)
    h = _gnorm(h, _pad_to(gn3_w, FW), _pad_to(gn3_b, FW), _pad_to(gn3_ms, FW))
    h = _cheb_layer(h, dis, row2, colp, W4p, b4p, act="relu")
    h = _cheb_layer(h, dis, row2, colp, W5p, b5p, act="relu")
    h = _cheb_layer(h, dis, row2, colp, W6p, b6p)
    h = _gnorm(h, _pad_to(gn6_w, FW), _pad_to(gn6_b, FW), _pad_to(gn6_ms, FW))

    st = _colstats(h)
    pooled = st[0] / N_NODES
    pooled8 = jnp.broadcast_to(pooled[None], (8, h.shape[1]))
    lin1p = jnp.pad(lin1_W, ((0, FW - lin1_W.shape[0]), (0, 0)))
    out = _head(pooled8, lin1p, lin1_b, lin2_W, lin2_b)
    return out[0:1]
